# Initial kernel scaffold; baseline (speedup 1.0000x reference)
import jax
import jax.numpy as jnp
from jax.experimental import pallas as pl


def kernel(nodes, neighbors, emb_table, W0a, b0a, g0a, be0a, W0b, b0b, g0, be0, W1a, b1a, g1a, be1a, W1b, b1b, g1, be1):
    raise NotImplementedError("write your pallas kernel here")


# speedup vs baseline: 15.5916x; 15.5916x over previous
import functools

import jax
import jax.numpy as jnp
from jax import lax
from jax.experimental import pallas as pl
from jax.experimental.pallas import tpu as pltpu
from jax.experimental.pallas import tpu_sc as plsc

_N = 10000
_DIN = 10000
_D = 128
_K = 5

_NC = 2
_NS = 16
_L = 16
_NW = _NC * _NS
_BW = 320
_PAD_N = _NW * _BW
_SB = 64
_NSB = _BW // _SB
_J = _K + 1


def _proj_body(emb_ref, w_ref, out_ref):
    out_ref[...] = jnp.dot(emb_ref[...], w_ref[...],
                           preferred_element_type=jnp.float32)


def _project(emb, w):
    bn = 400
    return pl.pallas_call(
        _proj_body,
        grid=(_DIN // bn,),
        in_specs=[
            pl.BlockSpec((bn, _DIN), lambda i: (i, 0)),
            pl.BlockSpec((_DIN, _D), lambda i: (0, 0)),
        ],
        out_specs=pl.BlockSpec((bn, _D), lambda i: (i, 0)),
        out_shape=jax.ShapeDtypeStruct((_DIN, _D), jnp.float32),
    )(emb, w)


def _agg_body(p_hbm, idx_hbm, out_hbm, *rest):
    idx_vs = rest[:_J]
    bufs_v, acc_v, sem = rest[_J:]
    wid = lax.axis_index("s") * _NC + lax.axis_index("c")
    base = wid * _BW
    for j in range(_J):
        pltpu.sync_copy(idx_hbm.at[pl.ds(j * _PAD_N + base, _BW)], idx_vs[j])

    for sb in range(_NSB):
        off = sb * _SB
        cps = []
        for j in range(_J):
            cps.append(
                pltpu.async_copy(p_hbm.at[idx_vs[j].at[pl.ds(off, _SB)]],
                                 bufs_v.at[j], sem))
        for cp in cps:
            cp.wait()

        def _acc(r, carry):
            for c in range(_D // _L):
                s = pl.ds(c * _L, _L)
                v = bufs_v[0, r, s]
                v = v + v
                for j in range(1, _J):
                    v = v + bufs_v[j, r, s]
                acc_v[r, s] = v
            return carry

        lax.fori_loop(0, _SB, _acc, 0)
        pltpu.sync_copy(acc_v, out_hbm.at[pl.ds(base + off, _SB)])


def _aggregate(p, idx):
    mesh = plsc.VectorSubcoreMesh(core_axis_name="c", subcore_axis_name="s")
    fn = functools.partial(
        pl.kernel,
        mesh=mesh,
        out_type=jax.ShapeDtypeStruct((_PAD_N, _D), jnp.float32),
        scratch_types=[pltpu.VMEM((_BW,), jnp.int32) for _ in range(_J)] + [
            pltpu.VMEM((_J, _SB, _D), jnp.float32),
            pltpu.VMEM((_SB, _D), jnp.float32),
            pltpu.SemaphoreType.DMA,
        ],
    )(_agg_body)
    return fn(p, idx)


def _bn_relu(x, g, b):
    mu = jnp.mean(x, axis=0, keepdims=True)
    var = jnp.mean((x - mu) * (x - mu), axis=0, keepdims=True)
    y = g * (x - mu) / jnp.sqrt(var + 1e-5) + b
    return jnp.maximum(y, 0.0)


def _mlp_body(agg_ref, g0a_ref, be0a_ref, w0b_ref, g0_ref, be0_ref,
              w1a_ref, g1a_ref, be1a_ref, w1b_ref, g1_ref, be1_ref, out_ref):
    h = _bn_relu(agg_ref[...], g0a_ref[...], be0a_ref[...])
    h = jnp.dot(h, w0b_ref[...], preferred_element_type=jnp.float32)
    h = _bn_relu(h, g0_ref[...], be0_ref[...])
    h = jnp.dot(h, w1a_ref[...], preferred_element_type=jnp.float32)
    h = _bn_relu(h, g1a_ref[...], be1a_ref[...])
    h = jnp.dot(h, w1b_ref[...], preferred_element_type=jnp.float32)
    h = _bn_relu(h, g1_ref[...], be1_ref[...])
    out_ref[...] = h.T


def _mlp(agg, g0a, be0a, w0b, g0, be0, w1a, g1a, be1a, w1b, g1, be1):
    row = lambda v: v.reshape(1, _D)
    return pl.pallas_call(
        _mlp_body,
        out_shape=jax.ShapeDtypeStruct((_D, _N), jnp.float32),
    )(agg, row(g0a), row(be0a), w0b, row(g0), row(be0),
      w1a, row(g1a), row(be1a), w1b, row(g1), row(be1))


def kernel(nodes, neighbors, emb_table, W0a, b0a, g0a, be0a, W0b, b0b, g0,
           be0, W1a, b1a, g1a, be1a, W1b, b1b, g1, be1):
    p = _project(emb_table, W0a)
    nb = jnp.take(neighbors, nodes, axis=0)
    idx = jnp.concatenate([nodes[None, :], nb.T], axis=0)
    idx = jnp.pad(idx, ((0, 0), (0, _PAD_N - _N))).reshape(-1)
    agg = _aggregate(p, idx)[:_N]
    return _mlp(agg, g0a, be0a, W0b, g0, be0, W1a, g1a, be1a, W1b, g1, be1)
